# Initial kernel scaffold; baseline (speedup 1.0000x reference)
import jax
import jax.numpy as jnp
from jax.experimental import pallas as pl


def kernel(x, t, noise, params):
    raise NotImplementedError("write your pallas kernel here")


# speedup vs baseline: 1.3693x; 1.3693x over previous
import jax
import jax.numpy as jnp
import numpy as np
from jax.experimental import pallas as pl
from jax.experimental.pallas import tpu as pltpu

_DIM = 512
_F32 = jnp.float32

_BETA = np.linspace(1e-4, 0.02, 1000)
_AC = np.cumprod(1.0 - _BETA)
_AC_TAB = np.zeros((8, 128), np.float32)
_AC_TAB.reshape(-1)[:1000] = _AC.astype(np.float32)

_FREQS = np.exp(-np.linspace(0.0, 10.0, _DIM // 2)).astype(np.float32)


def _conv3x3(xpad, wt, b, T, relu=False, stats=False, blend=None):
    N, Hp, Wp, Cin = xpad.shape
    H, W = Hp - 2, Wp - 2
    Co = wt.shape[-1]
    M = T * W

    def body(*refs):
        if blend is None:
            x_hbm, w_ref, b_ref = refs[:3]
            refs = refs[3:]
        else:
            x_hbm, n_hbm, t_ref, ac_ref, w_ref, b_ref = refs[:6]
            refs = refs[6:]
        if stats:
            y_ref, s_ref, q_ref = refs[:3]
            refs = refs[3:]
        else:
            y_ref = refs[0]
            refs = refs[1:]
        if blend is None:
            win, sem = refs
        else:
            win, nwin, sem, sem2 = refs

        n = pl.program_id(0)
        ht = pl.program_id(1)
        cp = pltpu.make_async_copy(x_hbm.at[n, pl.ds(ht * T, T + 2)], win, sem)
        cp.start()
        if blend is not None:
            cp2 = pltpu.make_async_copy(
                n_hbm.at[n, pl.ds(ht * T, T + 2)], nwin, sem2)
            cp2.start()
        cp.wait()
        if blend is None:
            wv = win[...]
        else:
            cp2.wait()
            idx = t_ref[n]
            io = (jax.lax.broadcasted_iota(jnp.int32, (8, 128), 0) * 128
                  + jax.lax.broadcasted_iota(jnp.int32, (8, 128), 1))
            acp = jnp.sum(jnp.where(io == idx, ac_ref[...], 0.0))
            sa = jnp.sqrt(acp)
            so = jnp.sqrt(jnp.maximum(1.0 - acp, 0.0))
            wv = sa * win[...] + so * nwin[...]

        acc = jnp.zeros((M, Co), _F32)
        for k in range(9):
            dy, dx = k // 3, k % 3
            m = wv[dy:dy + T, dx:dx + W, :].reshape(M, Cin)
            acc = acc + jnp.dot(m, w_ref[k], preferred_element_type=_F32)
        y = acc + b_ref[0][None, :]
        if relu:
            y = jnp.maximum(y, 0.0)
        y_ref[...] = y.reshape(1, T, W, Co)
        if stats:
            @pl.when(jnp.logical_and(n == 0, ht == 0))
            def _():
                s_ref[...] = jnp.zeros((8, Co), _F32)
                q_ref[...] = jnp.zeros((8, Co), _F32)
            s_ref[...] += y.reshape(8, M // 8, Co).sum(axis=1)
            q_ref[...] += (y * y).reshape(8, M // 8, Co).sum(axis=1)

    in_specs = [pl.BlockSpec(memory_space=pl.ANY)]
    inputs = [xpad]
    scratch = [pltpu.VMEM((T + 2, Wp, Cin), _F32), pltpu.SemaphoreType.DMA]
    if blend is not None:
        noise_pad, t, ac_tab = blend
        in_specs += [
            pl.BlockSpec(memory_space=pl.ANY),
            pl.BlockSpec(memory_space=pltpu.SMEM),
            pl.BlockSpec((8, 128), lambda n, h: (0, 0)),
        ]
        inputs += [noise_pad, t, ac_tab]
        scratch = [pltpu.VMEM((T + 2, Wp, Cin), _F32),
                   pltpu.VMEM((T + 2, Wp, Cin), _F32),
                   pltpu.SemaphoreType.DMA, pltpu.SemaphoreType.DMA]
    in_specs += [
        pl.BlockSpec((9, Cin, Co), lambda n, h: (0, 0, 0)),
        pl.BlockSpec((1, Co), lambda n, h: (0, 0)),
    ]
    inputs += [wt, b]

    out_shape = [jax.ShapeDtypeStruct((N, H, W, Co), _F32)]
    out_specs = [pl.BlockSpec((1, T, W, Co), lambda n, h: (n, h, 0, 0))]
    if stats:
        out_shape += [jax.ShapeDtypeStruct((8, Co), _F32)] * 2
        out_specs += [pl.BlockSpec((8, Co), lambda n, h: (0, 0))] * 2

    return pl.pallas_call(
        body,
        grid=(N, H // T),
        in_specs=in_specs,
        out_specs=out_specs,
        out_shape=out_shape,
        scratch_shapes=scratch,
    )(*inputs)


def _bn_relu_pool(y, s, q, T, pool):
    N, H, W, C = y.shape
    cnt = float(N * H * W)
    M = T * W

    def body(y_ref, s_ref, q_ref, e_ref, *rest):
        if pool:
            p_ref, f_ref = rest
        else:
            f_ref, = rest
        ht = pl.program_id(1)
        mean = jnp.sum(s_ref[...], axis=0) / cnt
        var = jnp.sum(q_ref[...], axis=0) / cnt - mean * mean
        inv = jax.lax.rsqrt(var + 1e-5)
        e = jnp.maximum((y_ref[0] - mean[None, None, :]) * inv[None, None, :],
                        0.0)
        e_ref[...] = e[None]
        if pool:
            ph = jnp.max(e.reshape(T // 2, 2, W, C), axis=1)
            pw = jnp.max(ph.reshape(T // 2, W // 2, 2, C), axis=2)
            p_ref[...] = pw[None]

        @pl.when(ht == 0)
        def _():
            f_ref[...] = jnp.zeros((1, 8, C), _F32)
        f_ref[...] += (e.reshape(8, M // 8, C).sum(axis=1) / float(H * W))[None]

    out_shape = [jax.ShapeDtypeStruct((N, H, W, C), _F32)]
    out_specs = [pl.BlockSpec((1, T, W, C), lambda n, h: (n, h, 0, 0))]
    if pool:
        out_shape.append(jax.ShapeDtypeStruct((N, H // 2, W // 2, C), _F32))
        out_specs.append(
            pl.BlockSpec((1, T // 2, W // 2, C), lambda n, h: (n, h, 0, 0)))
    out_shape.append(jax.ShapeDtypeStruct((N, 8, C), _F32))
    out_specs.append(pl.BlockSpec((1, 8, C), lambda n, h: (n, 0, 0)))

    return pl.pallas_call(
        body,
        grid=(N, H // T),
        in_specs=[
            pl.BlockSpec((1, T, W, C), lambda n, h: (n, h, 0, 0)),
            pl.BlockSpec((8, C), lambda n, h: (0, 0)),
            pl.BlockSpec((8, C), lambda n, h: (0, 0)),
        ],
        out_specs=out_specs,
        out_shape=out_shape,
    )(y, s, q)


def _up_weights(n):
    ys = np.linspace(0.0, n - 1.0, 2 * n)
    y0 = np.floor(ys).astype(np.int64)
    wy = (ys - y0).astype(np.float32)
    we = wy[0::2]
    wo = wy[1::2]
    return np.stack([1.0 - we, we, 1.0 - wo, wo], axis=1)


def _up_hmat(h, T2, Th, Thw):
    H2 = 2 * h
    ys = np.linspace(0.0, h - 1.0, H2)
    y0 = np.floor(ys).astype(np.int64)
    y1 = np.minimum(y0 + 1, h - 1)
    wy = (ys - y0).astype(np.float64)
    HT = H2 // T2
    mats = np.zeros((HT, T2, Thw), np.float32)
    bases = np.zeros((HT,), np.int64)
    for ht in range(HT):
        r0 = min(max(ht * Th - 1, 0), h - Thw)
        bases[ht] = r0
        for i in range(T2):
            g = ht * T2 + i
            mats[ht, i, y0[g] - r0] += 1.0 - wy[g]
            mats[ht, i, y1[g] - r0] += wy[g]
    return mats, bases


def _upsample2(x, T2=56):
    N, h, w, C = x.shape
    Th = T2 // 2
    Thw = min(Th + 2, h)
    mats, _ = _up_hmat(h, T2, Th, Thw)
    uh = jnp.asarray(mats)
    ww = jnp.asarray(_up_weights(w))

    def body(x_hbm, uh_ref, ww_ref, o_ref, win, sem):
        n = pl.program_id(0)
        ht = pl.program_id(1)
        r0 = jnp.clip(ht * Th - 1, 0, h - Thw)
        cp = pltpu.make_async_copy(x_hbm.at[n, pl.ds(r0, Thw)], win, sem)
        cp.start()
        cp.wait()
        rows = jnp.dot(uh_ref[0], win[...].reshape(Thw, w * C),
                       preferred_element_type=_F32).reshape(T2, w, C)

        ae2 = ww_ref[:, 0].reshape(1, w, 1)
        be2 = ww_ref[:, 1].reshape(1, w, 1)
        ao2 = ww_ref[:, 2].reshape(1, w, 1)
        bo2 = ww_ref[:, 3].reshape(1, w, 1)
        tm = jnp.concatenate([rows[:, :1], rows[:, :-1]], axis=1)
        tp = jnp.concatenate([rows[:, 1:], rows[:, -1:]], axis=1)
        ev2 = ae2 * tm + be2 * rows
        od2 = ao2 * rows + bo2 * tp
        o_ref[...] = jnp.stack([ev2, od2], axis=2).reshape(1, T2, 2 * w, C)

    return pl.pallas_call(
        body,
        grid=(N, (2 * h) // T2),
        in_specs=[
            pl.BlockSpec(memory_space=pl.ANY),
            pl.BlockSpec((1, T2, Thw), lambda n, t: (t, 0, 0)),
            pl.BlockSpec((w, 4), lambda n, t: (0, 0)),
        ],
        out_specs=pl.BlockSpec((1, T2, 2 * w, C), lambda n, t: (n, t, 0, 0)),
        out_shape=jax.ShapeDtypeStruct((N, 2 * h, 2 * w, C), _F32),
        scratch_shapes=[pltpu.VMEM((Thw, w, C), _F32),
                        pltpu.SemaphoreType.DMA],
    )(x, uh, ww)


def _time_embed(tf, w1, b1, w2, b2):
    N = tf.shape[0]
    freqs = jnp.asarray(_FREQS).reshape(1, _DIM // 2)

    def body(t_ref, f_ref, w1_ref, b1_ref, w2_ref, b2_ref, o_ref):
        s = jnp.sin(t_ref[...] * f_ref[...])
        emb = jnp.concatenate([s, jnp.cos(s)], axis=1)
        h = jnp.dot(emb, w1_ref[...], preferred_element_type=_F32) + b1_ref[...]
        h = h * jax.nn.sigmoid(h)
        o_ref[...] = (jnp.dot(h, w2_ref[...], preferred_element_type=_F32)
                      + b2_ref[...])

    return pl.pallas_call(
        body,
        out_shape=jax.ShapeDtypeStruct((N, _DIM), _F32),
    )(tf, freqs, w1, b1.reshape(1, -1), w2, b2.reshape(1, -1))


def _feats_cat(parts, scales):
    N = parts[0].shape[0]
    Ctot = sum(p.shape[-1] for p in parts)

    def body(*refs):
        o_ref = refs[-1]
        o_ref[...] = jnp.concatenate(
            [r[...].sum(axis=1) for r in refs[:-1]], axis=1)

    return pl.pallas_call(
        body,
        out_shape=jax.ShapeDtypeStruct((N, Ctot), _F32),
    )(*parts)


def _pad(a):
    return jnp.pad(a, ((0, 0), (1, 1), (1, 1), (0, 0)))


def _wt(w):
    return jnp.transpose(w, (2, 3, 1, 0)).reshape(9, w.shape[1], w.shape[0])


def kernel(x, t, noise, params):
    p = params
    t32 = t.astype(jnp.int32)
    tf = t.astype(_F32).reshape(-1, 1)
    ac_tab = jnp.asarray(_AC_TAB)

    xh = jnp.transpose(x, (0, 2, 3, 1))
    nh = jnp.transpose(noise, (0, 2, 3, 1))

    y1, s1, q1 = _conv3x3(_pad(xh), _wt(p['enc1_w']),
                          p['enc1_b'].reshape(1, -1), T=28, stats=True,
                          blend=(_pad(nh), t32, ac_tab))
    e1, p1, f1 = _bn_relu_pool(y1, s1, q1, T=28, pool=True)

    y2, s2, q2 = _conv3x3(_pad(p1), _wt(p['enc2_w']),
                          p['enc2_b'].reshape(1, -1), T=28, stats=True)
    e2, p2, f2 = _bn_relu_pool(y2, s2, q2, T=28, pool=True)

    y3, s3, q3 = _conv3x3(_pad(p2), _wt(p['enc3_w']),
                          p['enc3_b'].reshape(1, -1), T=28, stats=True)
    e3, p3, f3 = _bn_relu_pool(y3, s3, q3, T=28, pool=True)

    y4, s4, q4 = _conv3x3(_pad(p3), _wt(p['enc4_w']),
                          p['enc4_b'].reshape(1, -1), T=28, stats=True)
    e4, f4 = _bn_relu_pool(y4, s4, q4, T=28, pool=False)

    u4 = _upsample2(e4)
    d1 = _conv3x3(_pad(jnp.concatenate([u4, e3], axis=-1)),
                  _wt(p['dec1_w']), p['dec1_b'].reshape(1, -1), T=14,
                  relu=True)[0]
    u1 = _upsample2(d1)
    d2 = _conv3x3(_pad(jnp.concatenate([u1, e2], axis=-1)),
                  _wt(p['dec2_w']), p['dec2_b'].reshape(1, -1), T=28,
                  relu=True)[0]
    u2 = _upsample2(d2)
    d3 = _conv3x3(_pad(jnp.concatenate([u2, e1], axis=-1)),
                  _wt(p['dec3_w']), p['dec3_b'].reshape(1, -1), T=28,
                  relu=True)[0]
    out = _conv3x3(_pad(d3), _wt(p['dec4_w']), p['dec4_b'].reshape(1, -1),
                   T=28)[0]

    noise_pred = jnp.transpose(out, (0, 3, 1, 2))
    feats = _feats_cat([f1, f2, f3, f4], None)
    t_emb = _time_embed(tf, p['te_w1'], p['te_b1'], p['te_w2'], p['te_b2'])
    return (noise_pred, feats, t_emb)


# speedup vs baseline: 1.5561x; 1.1364x over previous
import jax
import jax.numpy as jnp
import numpy as np
from jax.experimental import pallas as pl
from jax.experimental.pallas import tpu as pltpu

_DIM = 512
_F32 = jnp.float32

_BETA = np.linspace(1e-4, 0.02, 1000)
_AC = np.cumprod(1.0 - _BETA)
_AC_TAB = np.zeros((8, 128), np.float32)
_AC_TAB.reshape(-1)[:1000] = _AC.astype(np.float32)

_FREQS = np.exp(-np.linspace(0.0, 10.0, _DIM // 2)).astype(np.float32)


def _conv3x3(xpad, wt, b, T, relu=False, stats=False, blend=None):
    N, Hp, Wp, Cin = xpad.shape
    H, W = Hp - 2, Wp - 2
    Co = wt.shape[-1]
    M = T * W

    def body(*refs):
        if blend is None:
            x_hbm, w_ref, b_ref = refs[:3]
            refs = refs[3:]
        else:
            x_hbm, n_hbm, t_ref, ac_ref, w_ref, b_ref = refs[:6]
            refs = refs[6:]
        if stats:
            y_ref, s_ref, q_ref = refs[:3]
            refs = refs[3:]
        else:
            y_ref = refs[0]
            refs = refs[1:]
        if blend is None:
            win, sem = refs
        else:
            win, nwin, sem, sem2 = refs

        n = pl.program_id(0)
        ht = pl.program_id(1)
        cp = pltpu.make_async_copy(x_hbm.at[n, pl.ds(ht * T, T + 2)], win, sem)
        cp.start()
        if blend is not None:
            cp2 = pltpu.make_async_copy(
                n_hbm.at[n, pl.ds(ht * T, T + 2)], nwin, sem2)
            cp2.start()
        cp.wait()
        if blend is None:
            wv = win[...]
        else:
            cp2.wait()
            idx = t_ref[n]
            io = (jax.lax.broadcasted_iota(jnp.int32, (8, 128), 0) * 128
                  + jax.lax.broadcasted_iota(jnp.int32, (8, 128), 1))
            acp = jnp.sum(jnp.where(io == idx, ac_ref[...], 0.0))
            sa = jnp.sqrt(acp)
            so = jnp.sqrt(jnp.maximum(1.0 - acp, 0.0))
            wv = sa * win[...] + so * nwin[...]

        acc = jnp.zeros((M, Co), _F32)
        for k in range(9):
            dy, dx = k // 3, k % 3
            m = wv[dy:dy + T, dx:dx + W, :].reshape(M, Cin)
            acc = acc + jnp.dot(m, w_ref[k], preferred_element_type=_F32)
        y = acc + b_ref[0][None, :]
        if relu:
            y = jnp.maximum(y, 0.0)
        y_ref[...] = y.reshape(1, T, W, Co)
        if stats:
            @pl.when(jnp.logical_and(n == 0, ht == 0))
            def _():
                s_ref[...] = jnp.zeros((8, Co), _F32)
                q_ref[...] = jnp.zeros((8, Co), _F32)
            s_ref[...] += y.reshape(8, M // 8, Co).sum(axis=1)
            q_ref[...] += (y * y).reshape(8, M // 8, Co).sum(axis=1)

    in_specs = [pl.BlockSpec(memory_space=pl.ANY)]
    inputs = [xpad]
    scratch = [pltpu.VMEM((T + 2, Wp, Cin), _F32), pltpu.SemaphoreType.DMA]
    if blend is not None:
        noise_pad, t, ac_tab = blend
        in_specs += [
            pl.BlockSpec(memory_space=pl.ANY),
            pl.BlockSpec(memory_space=pltpu.SMEM),
            pl.BlockSpec((8, 128), lambda n, h: (0, 0)),
        ]
        inputs += [noise_pad, t, ac_tab]
        scratch = [pltpu.VMEM((T + 2, Wp, Cin), _F32),
                   pltpu.VMEM((T + 2, Wp, Cin), _F32),
                   pltpu.SemaphoreType.DMA, pltpu.SemaphoreType.DMA]
    in_specs += [
        pl.BlockSpec((9, Cin, Co), lambda n, h: (0, 0, 0)),
        pl.BlockSpec((1, Co), lambda n, h: (0, 0)),
    ]
    inputs += [wt, b]

    out_shape = [jax.ShapeDtypeStruct((N, H, W, Co), _F32)]
    out_specs = [pl.BlockSpec((1, T, W, Co), lambda n, h: (n, h, 0, 0))]
    if stats:
        out_shape += [jax.ShapeDtypeStruct((8, Co), _F32)] * 2
        out_specs += [pl.BlockSpec((8, Co), lambda n, h: (0, 0))] * 2

    return pl.pallas_call(
        body,
        grid=(N, H // T),
        in_specs=in_specs,
        out_specs=out_specs,
        out_shape=out_shape,
        scratch_shapes=scratch,
    )(*inputs)


def _bn_relu_pool(y, s, q, T, pool):
    N, H, W, C = y.shape
    cnt = float(N * H * W)
    M = T * W
    T2 = T // 2
    W2 = W // 2

    def body(y_ref, s_ref, q_ref, *refs):
        if pool:
            (ep_hbm, pp_hbm, f_ref, et, pt, zr, zrp, se, sp, sz) = refs
        else:
            e_ref, f_ref = refs
        ht = pl.program_id(1)
        n = pl.program_id(0)
        mean = jnp.sum(s_ref[...], axis=0) / cnt
        var = jnp.sum(q_ref[...], axis=0) / cnt - mean * mean
        inv = jax.lax.rsqrt(var + 1e-5)
        e = jnp.maximum((y_ref[0] - mean[None, None, :]) * inv[None, None, :],
                        0.0)
        if not pool:
            e_ref[...] = e[None]
        else:
            et[:, 1:W + 1, :] = e
            et[:, 0:1, :] = jnp.zeros((T, 1, C), _F32)
            et[:, W + 1:W + 2, :] = jnp.zeros((T, 1, C), _F32)
            cpe = pltpu.make_async_copy(
                et, ep_hbm.at[n, pl.ds(1 + ht * T, T)], se)
            cpe.start()

            ph = jnp.max(e.reshape(T2, 2, W, C), axis=1)
            pw = jnp.max(ph.reshape(T2, W2, 2, C), axis=2)
            pt[:, 1:W2 + 1, :] = pw
            pt[:, 0:1, :] = jnp.zeros((T2, 1, C), _F32)
            pt[:, W2 + 1:W2 + 2, :] = jnp.zeros((T2, 1, C), _F32)
            cpp = pltpu.make_async_copy(
                pt, pp_hbm.at[n, pl.ds(1 + ht * T2, T2)], sp)
            cpp.start()

            @pl.when(ht == 0)
            def _():
                zr[...] = jnp.zeros((1, W + 2, C), _F32)
                zrp[...] = jnp.zeros((1, W2 + 2, C), _F32)
                for hbm, zsrc, last in ((ep_hbm, zr, H + 1),
                                        (pp_hbm, zrp, W2 * 0 + H // 2 + 1)):
                    for row in (0, last):
                        cz = pltpu.make_async_copy(
                            zsrc, hbm.at[n, pl.ds(row, 1)], sz)
                        cz.start()
                        cz.wait()
            cpe.wait()
            cpp.wait()

        @pl.when(ht == 0)
        def _():
            f_ref[...] = jnp.zeros((1, 8, C), _F32)
        f_ref[...] += (e.reshape(8, M // 8, C).sum(axis=1) / float(H * W))[None]

    in_specs = [
        pl.BlockSpec((1, T, W, C), lambda n, h: (n, h, 0, 0)),
        pl.BlockSpec((8, C), lambda n, h: (0, 0)),
        pl.BlockSpec((8, C), lambda n, h: (0, 0)),
    ]
    if pool:
        out_shape = [jax.ShapeDtypeStruct((N, H + 2, W + 2, C), _F32),
                     jax.ShapeDtypeStruct((N, H // 2 + 2, W // 2 + 2, C), _F32),
                     jax.ShapeDtypeStruct((N, 8, C), _F32)]
        out_specs = [pl.BlockSpec(memory_space=pl.ANY),
                     pl.BlockSpec(memory_space=pl.ANY),
                     pl.BlockSpec((1, 8, C), lambda n, h: (n, 0, 0))]
        scratch = [pltpu.VMEM((T, W + 2, C), _F32),
                   pltpu.VMEM((T2, W2 + 2, C), _F32),
                   pltpu.VMEM((1, W + 2, C), _F32),
                   pltpu.VMEM((1, W2 + 2, C), _F32),
                   pltpu.SemaphoreType.DMA, pltpu.SemaphoreType.DMA,
                   pltpu.SemaphoreType.DMA]
    else:
        out_shape = [jax.ShapeDtypeStruct((N, H, W, C), _F32),
                     jax.ShapeDtypeStruct((N, 8, C), _F32)]
        out_specs = [pl.BlockSpec((1, T, W, C), lambda n, h: (n, h, 0, 0)),
                     pl.BlockSpec((1, 8, C), lambda n, h: (n, 0, 0))]
        scratch = []

    return pl.pallas_call(
        body,
        grid=(N, H // T),
        in_specs=in_specs,
        out_specs=out_specs,
        out_shape=out_shape,
        scratch_shapes=scratch,
    )(y, s, q)


def _up_weights(n):
    ys = np.linspace(0.0, n - 1.0, 2 * n)
    y0 = np.floor(ys).astype(np.int64)
    wy = (ys - y0).astype(np.float32)
    we = wy[0::2]
    wo = wy[1::2]
    return np.stack([1.0 - we, we, 1.0 - wo, wo], axis=1)


def _up_hmat_dec(h, T, Ta):
    H2 = 2 * h
    ys = np.linspace(0.0, h - 1.0, H2)
    y0 = np.floor(ys).astype(np.int64)
    y1 = np.minimum(y0 + 1, h - 1)
    wy = ys - y0
    HT = H2 // T
    mats = np.zeros((HT, T + 2, Ta), np.float32)
    for ht in range(HT):
        t0 = ht * T
        base = min(max((t0 - 1) // 2 - 1, 0), h - Ta)
        for i in range(T + 2):
            r = t0 - 1 + i
            if r < 0 or r >= H2:
                continue
            mats[ht, i, y0[r] - base] += 1.0 - wy[r]
            mats[ht, i, y1[r] - base] += wy[r]
    return mats


def _dec_conv(a, b_pad, wt, b, T, pad_out=False):
    N, h, w, Ca = a.shape
    Cb = b_pad.shape[-1] if b_pad is not None else 0
    H2, W2 = 2 * h, 2 * w
    Co = wt.shape[-1]
    M = T * W2
    Ta = T // 2 + 3
    uh = jnp.asarray(_up_hmat_dec(h, T, Ta))
    ww = jnp.asarray(_up_weights(w))
    wta = wt[:, :Ca]
    wtb = wt[:, Ca:]

    def body(a_hbm, bp_hbm, uh_ref, ww_ref, wa_ref, wb_ref, b_ref,
             o_ref, awin, bwin, sa_, sb_, *orest):
        n = pl.program_id(0)
        ht = pl.program_id(1)
        t0 = ht * T
        base = jnp.minimum(jnp.maximum((t0 - 1) // 2 - 1, 0), h - Ta)
        cpa = pltpu.make_async_copy(a_hbm.at[n, pl.ds(base, Ta)], awin, sa_)
        cpa.start()
        cpb = pltpu.make_async_copy(bp_hbm.at[n, pl.ds(t0, T + 2)], bwin, sb_)
        cpb.start()
        cpa.wait()

        rows = jnp.dot(uh_ref[0], awin[...].reshape(Ta, w * Ca),
                       preferred_element_type=_F32).reshape(T + 2, w, Ca)
        ae2 = ww_ref[:, 0].reshape(1, w, 1)
        be2 = ww_ref[:, 1].reshape(1, w, 1)
        ao2 = ww_ref[:, 2].reshape(1, w, 1)
        bo2 = ww_ref[:, 3].reshape(1, w, 1)
        tm = jnp.concatenate([rows[:, :1], rows[:, :-1]], axis=1)
        tp = jnp.concatenate([rows[:, 1:], rows[:, -1:]], axis=1)
        ev2 = ae2 * tm + be2 * rows
        od2 = ao2 * rows + bo2 * tp
        up = jnp.stack([ev2, od2], axis=2).reshape(T + 2, W2, Ca)
        zc = jnp.zeros((T + 2, 1, Ca), _F32)
        upw = jnp.concatenate([zc, up, zc], axis=1)

        cpb.wait()
        bv = bwin[...]
        acc = jnp.zeros((M, Co), _F32)
        for k in range(9):
            dy, dx = k // 3, k % 3
            ma = upw[dy:dy + T, dx:dx + W2, :].reshape(M, Ca)
            acc = acc + jnp.dot(ma, wa_ref[k], preferred_element_type=_F32)
            mb = bv[dy:dy + T, dx:dx + W2, :].reshape(M, Cb)
            acc = acc + jnp.dot(mb, wb_ref[k], preferred_element_type=_F32)
        y = jnp.maximum(acc + b_ref[0][None, :], 0.0)
        if not pad_out:
            o_ref[...] = y.reshape(1, T, W2, Co)
        else:
            ot, zr, so_ = orest
            ot[:, 1:W2 + 1, :] = y.reshape(T, W2, Co)
            ot[:, 0:1, :] = jnp.zeros((T, 1, Co), _F32)
            ot[:, W2 + 1:W2 + 2, :] = jnp.zeros((T, 1, Co), _F32)
            cpo = pltpu.make_async_copy(
                ot, o_ref.at[n, pl.ds(1 + t0, T)], so_)
            cpo.start()

            @pl.when(ht == 0)
            def _():
                zr[...] = jnp.zeros((1, W2 + 2, Co), _F32)
                for row in (0, H2 + 1):
                    cz = pltpu.make_async_copy(
                        zr, o_ref.at[n, pl.ds(row, 1)], so_)
                    cz.start()
                    cz.wait()
            cpo.wait()

    in_specs = [
        pl.BlockSpec(memory_space=pl.ANY),
        pl.BlockSpec(memory_space=pl.ANY),
        pl.BlockSpec((1, T + 2, Ta), lambda n, t: (t, 0, 0)),
        pl.BlockSpec((w, 4), lambda n, t: (0, 0)),
        pl.BlockSpec((9, Ca, Co), lambda n, t: (0, 0, 0)),
        pl.BlockSpec((9, Cb, Co), lambda n, t: (0, 0, 0)),
        pl.BlockSpec((1, Co), lambda n, t: (0, 0)),
    ]
    scratch = [pltpu.VMEM((Ta, w, Ca), _F32),
               pltpu.VMEM((T + 2, W2 + 2, Cb), _F32),
               pltpu.SemaphoreType.DMA, pltpu.SemaphoreType.DMA]
    if pad_out:
        out_shape = jax.ShapeDtypeStruct((N, H2 + 2, W2 + 2, Co), _F32)
        out_specs = pl.BlockSpec(memory_space=pl.ANY)
        scratch += [pltpu.VMEM((T, W2 + 2, Co), _F32),
                    pltpu.VMEM((1, W2 + 2, Co), _F32),
                    pltpu.SemaphoreType.DMA]
    else:
        out_shape = jax.ShapeDtypeStruct((N, H2, W2, Co), _F32)
        out_specs = pl.BlockSpec((1, T, W2, Co), lambda n, t: (n, t, 0, 0))

    return pl.pallas_call(
        body,
        grid=(N, H2 // T),
        in_specs=in_specs,
        out_specs=out_specs,
        out_shape=out_shape,
        scratch_shapes=scratch,
    )(a, b_pad, uh, ww, wta, wtb, b)


def _time_embed(tf, w1, b1, w2, b2):
    N = tf.shape[0]
    freqs = jnp.asarray(_FREQS).reshape(1, _DIM // 2)

    def body(t_ref, f_ref, w1_ref, b1_ref, w2_ref, b2_ref, o_ref):
        s = jnp.sin(t_ref[...] * f_ref[...])
        emb = jnp.concatenate([s, jnp.cos(s)], axis=1)
        h = jnp.dot(emb, w1_ref[...], preferred_element_type=_F32) + b1_ref[...]
        h = h * jax.nn.sigmoid(h)
        o_ref[...] = (jnp.dot(h, w2_ref[...], preferred_element_type=_F32)
                      + b2_ref[...])

    return pl.pallas_call(
        body,
        out_shape=jax.ShapeDtypeStruct((N, _DIM), _F32),
    )(tf, freqs, w1, b1.reshape(1, -1), w2, b2.reshape(1, -1))


def _feats_cat(parts):
    N = parts[0].shape[0]
    Ctot = sum(p.shape[-1] for p in parts)

    def body(*refs):
        o_ref = refs[-1]
        o_ref[...] = jnp.concatenate(
            [r[...].sum(axis=1) for r in refs[:-1]], axis=1)

    return pl.pallas_call(
        body,
        out_shape=jax.ShapeDtypeStruct((N, Ctot), _F32),
    )(*parts)


def _pad(a):
    return jnp.pad(a, ((0, 0), (1, 1), (1, 1), (0, 0)))


def _wt(w):
    return jnp.transpose(w, (2, 3, 1, 0)).reshape(9, w.shape[1], w.shape[0])


def kernel(x, t, noise, params):
    p = params
    t32 = t.astype(jnp.int32)
    tf = t.astype(_F32).reshape(-1, 1)
    ac_tab = jnp.asarray(_AC_TAB)

    xh = jnp.transpose(x, (0, 2, 3, 1))
    nh = jnp.transpose(noise, (0, 2, 3, 1))

    y1, s1, q1 = _conv3x3(_pad(xh), _wt(p['enc1_w']),
                          p['enc1_b'].reshape(1, -1), T=28, stats=True,
                          blend=(_pad(nh), t32, ac_tab))
    e1p, p1p, f1 = _bn_relu_pool(y1, s1, q1, T=28, pool=True)

    y2, s2, q2 = _conv3x3(p1p, _wt(p['enc2_w']),
                          p['enc2_b'].reshape(1, -1), T=28, stats=True)
    e2p, p2p, f2 = _bn_relu_pool(y2, s2, q2, T=28, pool=True)

    y3, s3, q3 = _conv3x3(p2p, _wt(p['enc3_w']),
                          p['enc3_b'].reshape(1, -1), T=28, stats=True)
    e3p, p3p, f3 = _bn_relu_pool(y3, s3, q3, T=28, pool=True)

    y4, s4, q4 = _conv3x3(p3p, _wt(p['enc4_w']),
                          p['enc4_b'].reshape(1, -1), T=28, stats=True)
    e4, f4 = _bn_relu_pool(y4, s4, q4, T=28, pool=False)

    d1 = _dec_conv(e4, e3p, _wt(p['dec1_w']), p['dec1_b'].reshape(1, -1),
                   T=28)
    d2 = _dec_conv(d1, e2p, _wt(p['dec2_w']), p['dec2_b'].reshape(1, -1),
                   T=28)
    d3p = _dec_conv(d2, e1p, _wt(p['dec3_w']), p['dec3_b'].reshape(1, -1),
                    T=28, pad_out=True)
    out = _conv3x3(d3p, _wt(p['dec4_w']), p['dec4_b'].reshape(1, -1), T=28)[0]

    noise_pred = jnp.transpose(out, (0, 3, 1, 2))
    feats = _feats_cat([f1, f2, f3, f4])
    t_emb = _time_embed(tf, p['te_w1'], p['te_b1'], p['te_w2'], p['te_b2'])
    return (noise_pred, feats, t_emb)
